# Initial kernel scaffold; baseline (speedup 1.0000x reference)
import functools

import jax
import jax.numpy as jnp
from jax import lax
from jax.experimental import pallas as pl
from jax.experimental.pallas import tpu as pltpu
from jax.experimental.pallas import tpu_sc as plsc

N = 10000
E = 320000
D = 128
H = 64
EPS = 1e-5

NC = 2
NS = 16
NW = NC * NS
CH = 128
EPW = E // NW
NFULL = EPW // CH
TAIL = EPW - NFULL * CH

_MESH = dict(core_axis_name="c", subcore_axis_name="s", num_cores=NC,
             num_subcores=NS)

ROWS_PER_SUB = N // NS


def _sc_gather(T, edge_index):
    @functools.partial(
        pl.kernel,
        out_type=jax.ShapeDtypeStruct((E, H), jnp.float32),
        mesh=plsc.VectorSubcoreMesh(**_MESH),
        scratch_types=[
            pltpu.VMEM((CH,), jnp.int32),
            pltpu.VMEM((CH,), jnp.int32),
            pltpu.VMEM((CH, H), jnp.float32),
            pltpu.VMEM((TAIL,), jnp.int32),
            pltpu.VMEM((TAIL,), jnp.int32),
            pltpu.VMEM((TAIL, H), jnp.float32),
        ],
    )
    def k(t_hbm, ei_hbm, pre_hbm, idxd, idxs, buf, idxd_t, idxs_t, buf_t):
        c = lax.axis_index("c")
        s = lax.axis_index("s")
        wid = s * NC + c

        def do_chunk(base, n, i_d, i_s, b):
            pltpu.sync_copy(ei_hbm.at[1, pl.ds(base, n)], i_d)
            pltpu.sync_copy(ei_hbm.at[0, pl.ds(base, n)], i_s)
            for j in range(n // 16):
                sl = pl.ds(j * 16, 16)
                i_s[sl] = i_s[sl] + N
            pltpu.sync_copy(t_hbm.at[i_d], b)
            pltpu.sync_copy(t_hbm.at[i_s], b, add=True)
            pltpu.sync_copy(b, pre_hbm.at[pl.ds(base, n)])

        def body(i, carry):
            do_chunk(wid * EPW + i * CH, CH, idxd, idxs, buf)
            return carry

        lax.fori_loop(0, NFULL, body, 0)
        do_chunk(wid * EPW + NFULL * CH, TAIL, idxd_t, idxs_t, buf_t)

    return k(T, edge_index)


def _sc_scatter(h2, edge_index, kap):
    @functools.partial(
        pl.kernel,
        out_type=jax.ShapeDtypeStruct((NC, N, H), jnp.float32),
        mesh=plsc.VectorSubcoreMesh(**_MESH),
        scratch_types=[
            pltpu.VMEM_SHARED((N, H), jnp.float32),
            pltpu.VMEM((ROWS_PER_SUB, H), jnp.float32),
            pltpu.VMEM((CH,), jnp.int32),
            pltpu.VMEM((CH, H), jnp.float32),
            pltpu.VMEM((TAIL,), jnp.int32),
            pltpu.VMEM((TAIL, H), jnp.float32),
            pltpu.VMEM((H,), jnp.float32),
        ],
    )
    def k(h2_hbm, ei_hbm, kap_hbm, out_hbm, acc, stage, idx, buf, idx_t,
          buf_t, kapv):
        c = lax.axis_index("c")
        s = lax.axis_index("s")
        wid = s * NC + c

        def zrow(i, carry):
            for j in range(H // 16):
                stage[i, pl.ds(j * 16, 16)] = jnp.zeros((16,), jnp.float32)
            return carry

        lax.fori_loop(0, ROWS_PER_SUB, zrow, 0)
        pltpu.sync_copy(stage, acc.at[pl.ds(s * ROWS_PER_SUB, ROWS_PER_SUB)])
        pltpu.sync_copy(kap_hbm, kapv)
        plsc.subcore_barrier()

        def do_chunk(base, n, i_r, b):
            pltpu.sync_copy(ei_hbm.at[1, pl.ds(base, n)], i_r)
            pltpu.sync_copy(h2_hbm.at[pl.ds(base, n)], b)

            def row(r, carry):
                for j in range(H // 16):
                    sl = pl.ds(j * 16, 16)
                    b[r, sl] = b[r, sl] + kapv[sl]
                return carry

            lax.fori_loop(0, n, row, 0)
            pltpu.sync_copy(b, acc.at[i_r], add=True)

        def body(i, carry):
            do_chunk(wid * EPW + i * CH, CH, idx, buf)
            return carry

        lax.fori_loop(0, NFULL, body, 0)
        do_chunk(wid * EPW + NFULL * CH, TAIL, idx_t, buf_t)

        plsc.subcore_barrier()
        sl = pl.ds(s * ROWS_PER_SUB, ROWS_PER_SUB)
        pltpu.sync_copy(acc.at[sl], stage)
        pltpu.sync_copy(stage, out_hbm.at[c, sl])

    return k(h2, edge_index, kap)


def _prep1(x, wcat):
    BR, NB = 2000, N // 2000

    def body(x_ref, w_ref, o_ref):
        o_ref[...] = jnp.dot(x_ref[...], w_ref[...],
                             preferred_element_type=jnp.float32)

    return pl.pallas_call(
        body,
        grid=(2, NB),
        in_specs=[
            pl.BlockSpec((BR, D), lambda p, j: (j, 0)),
            pl.BlockSpec((D, H), lambda p, j: (0, p)),
        ],
        out_specs=pl.BlockSpec((BR, H), lambda p, j: (p * NB + j, 0)),
        out_shape=jax.ShapeDtypeStruct((2 * N, H), jnp.float32),
    )(x, wcat)


def _prep2(S, wcat):
    BR, NB = 2000, N // 2000

    def body(s_ref, w_ref, o_ref):
        xin = s_ref[0] + s_ref[1]
        o_ref[...] = jnp.dot(xin, w_ref[...],
                             preferred_element_type=jnp.float32)

    return pl.pallas_call(
        body,
        grid=(2, NB),
        in_specs=[
            pl.BlockSpec((2, BR, H), lambda p, j: (0, j, 0)),
            pl.BlockSpec((H, H), lambda p, j: (0, p)),
        ],
        out_specs=pl.BlockSpec((BR, H), lambda p, j: (p * NB + j, 0)),
        out_shape=jax.ShapeDtypeStruct((2 * N, H), jnp.float32),
    )(S, wcat)


def _edge_stats(pre, b1row):
    BR = 4000
    NB = E // BR

    def body(p_ref, b_ref, o_ref):
        h = jnp.maximum(p_ref[...] + b_ref[...], 0.0)

        @pl.when(pl.program_id(0) == 0)
        def _():
            o_ref[...] = jnp.zeros_like(o_ref)

        o_ref[0:1, :] += jnp.sum(h, axis=0, keepdims=True)
        o_ref[1:2, :] += jnp.sum(h * h, axis=0, keepdims=True)

    return pl.pallas_call(
        body,
        grid=(NB,),
        in_specs=[
            pl.BlockSpec((BR, H), lambda i: (i, 0)),
            pl.BlockSpec((1, H), lambda i: (0, 0)),
        ],
        out_specs=pl.BlockSpec((8, H), lambda i: (0, 0)),
        out_shape=jax.ShapeDtypeStruct((8, H), jnp.float32),
    )(pre, b1row)


def _edge_mm(pre, b1row, w2, b2row):
    BR = 4000
    NB = E // BR

    def body(p_ref, b1_ref, w_ref, b2_ref, h2_ref, st_ref):
        h1 = jnp.maximum(p_ref[...] + b1_ref[...], 0.0)
        h2 = jnp.maximum(
            jnp.dot(h1, w_ref[...], preferred_element_type=jnp.float32)
            + b2_ref[...], 0.0)
        h2_ref[...] = h2

        @pl.when(pl.program_id(0) == 0)
        def _():
            st_ref[...] = jnp.zeros_like(st_ref)

        st_ref[0:1, :] += jnp.sum(h2, axis=0, keepdims=True)
        st_ref[1:2, :] += jnp.sum(h2 * h2, axis=0, keepdims=True)

    return pl.pallas_call(
        body,
        grid=(NB,),
        in_specs=[
            pl.BlockSpec((BR, H), lambda i: (i, 0)),
            pl.BlockSpec((1, H), lambda i: (0, 0)),
            pl.BlockSpec((H, H), lambda i: (0, 0)),
            pl.BlockSpec((1, H), lambda i: (0, 0)),
        ],
        out_specs=[
            pl.BlockSpec((BR, H), lambda i: (i, 0)),
            pl.BlockSpec((8, H), lambda i: (0, 0)),
        ],
        out_shape=[
            jax.ShapeDtypeStruct((E, H), jnp.float32),
            jax.ShapeDtypeStruct((8, H), jnp.float32),
        ],
    )(pre, b1row, w2, b2row)


def _cat_mm(S1, S2, w, brow):
    BR, NB = 2000, N // 2000
    MH = w.shape[1]

    def body(s1_ref, s2_ref, w_ref, b_ref, r_ref, st_ref):
        x1 = s1_ref[0] + s1_ref[1]
        x2 = s2_ref[0] + s2_ref[1]
        xcat = jnp.concatenate([x1, x2], axis=1)
        r = jnp.maximum(
            jnp.dot(xcat, w_ref[...], preferred_element_type=jnp.float32)
            + b_ref[...], 0.0)
        r_ref[...] = r

        @pl.when(pl.program_id(0) == 0)
        def _():
            st_ref[...] = jnp.zeros_like(st_ref)

        st_ref[0:1, :] += jnp.sum(r, axis=0, keepdims=True)
        st_ref[1:2, :] += jnp.sum(r * r, axis=0, keepdims=True)

    return pl.pallas_call(
        body,
        grid=(NB,),
        in_specs=[
            pl.BlockSpec((2, BR, H), lambda j: (0, j, 0)),
            pl.BlockSpec((2, BR, H), lambda j: (0, j, 0)),
            pl.BlockSpec((2 * H, MH), lambda j: (0, 0)),
            pl.BlockSpec((1, MH), lambda j: (0, 0)),
        ],
        out_specs=[
            pl.BlockSpec((BR, MH), lambda j: (j, 0)),
            pl.BlockSpec((8, MH), lambda j: (0, 0)),
        ],
        out_shape=[
            jax.ShapeDtypeStruct((N, MH), jnp.float32),
            jax.ShapeDtypeStruct((8, MH), jnp.float32),
        ],
    )(S1, S2, w, brow)


def _node_mm(xin, w, brow):
    BR, NB = 2000, N // 2000
    K, M = w.shape

    def body(x_ref, w_ref, b_ref, r_ref, st_ref):
        r = jnp.maximum(
            jnp.dot(x_ref[...], w_ref[...],
                    preferred_element_type=jnp.float32) + b_ref[...], 0.0)
        r_ref[...] = r

        @pl.when(pl.program_id(0) == 0)
        def _():
            st_ref[...] = jnp.zeros_like(st_ref)

        st_ref[0:1, :] += jnp.sum(r, axis=0, keepdims=True)
        st_ref[1:2, :] += jnp.sum(r * r, axis=0, keepdims=True)

    return pl.pallas_call(
        body,
        grid=(NB,),
        in_specs=[
            pl.BlockSpec((BR, K), lambda j: (j, 0)),
            pl.BlockSpec((K, M), lambda j: (0, 0)),
            pl.BlockSpec((1, M), lambda j: (0, 0)),
        ],
        out_specs=[
            pl.BlockSpec((BR, M), lambda j: (j, 0)),
            pl.BlockSpec((8, M), lambda j: (0, 0)),
        ],
        out_shape=[
            jax.ShapeDtypeStruct((N, M), jnp.float32),
            jax.ShapeDtypeStruct((8, M), jnp.float32),
        ],
    )(xin, w, brow)


def _final_mm(xin, w, brow):
    BR, NB = 2000, N // 2000
    K, M = w.shape

    def body(x_ref, w_ref, b_ref, o_ref):
        z = jnp.dot(x_ref[...], w_ref[...],
                    preferred_element_type=jnp.float32) + b_ref[...]
        m = jnp.max(z, axis=1, keepdims=True)
        lse = jnp.log(jnp.sum(jnp.exp(z - m), axis=1, keepdims=True)) + m
        o_ref[...] = z - lse

    return pl.pallas_call(
        body,
        grid=(NB,),
        in_specs=[
            pl.BlockSpec((BR, K), lambda j: (j, 0)),
            pl.BlockSpec((K, M), lambda j: (0, 0)),
            pl.BlockSpec((1, M), lambda j: (0, 0)),
        ],
        out_specs=pl.BlockSpec((BR, M), lambda j: (j, 0)),
        out_shape=jax.ShapeDtypeStruct((N, M), jnp.float32),
    )(xin, w, brow)


def _bn_fold(st, g, be):
    mu = st[0] / E
    var = st[1] / E - mu * mu
    s = g / jnp.sqrt(var + EPS)
    return mu, s, be - s * mu


def _bn_fold_n(st, g, be):
    mu = st[0] / N
    var = st[1] / N - mu * mu
    s = g / jnp.sqrt(var + EPS)
    return mu, s, be - s * mu


def _edge_layer(xin_T, edge_index, b1, g1, be1, W2, b2, g2, be2):
    pre = _sc_gather(xin_T, edge_index)
    st1 = _edge_stats(pre, b1.reshape(1, H))
    _, s1, c1 = _bn_fold(st1, g1, be1)
    w2p = s1[:, None] * W2
    b2p = c1 @ W2 + b2
    h2, st2 = _edge_mm(pre, b1.reshape(1, H), w2p, b2p.reshape(1, H))
    mu2 = st2[0] / E
    var2 = st2[1] / E - mu2 * mu2
    s2 = g2 / jnp.sqrt(var2 + EPS)
    kap = be2 / s2 - mu2
    S = _sc_scatter(h2, edge_index, kap)
    return S, s2


def kernel(x, edge_index, c1_W1, c1_b1, c1_g1, c1_be1, c1_W2, c1_b2, c1_g2,
           c1_be2, c2_W1, c2_b1, c2_g1, c2_be1, c2_W2, c2_b2, c2_g2, c2_be2,
           l1_W, l1_b, l1_g, l1_be, m1_W, m1_b, m1_g, m1_be, m2_W, m2_b,
           m2_g, m2_be, f_W, f_b):
    wcat1 = jnp.concatenate([c1_W1[:D] - c1_W1[D:], c1_W1[D:]], axis=1)
    T1 = _prep1(x, wcat1)
    S1, s2a = _edge_layer(T1, edge_index, c1_b1, c1_g1, c1_be1,
                          c1_W2, c1_b2, c1_g2, c1_be2)

    wcat2 = s2a[:, None] * jnp.concatenate(
        [c2_W1[:H] - c2_W1[H:], c2_W1[H:]], axis=1)
    T2 = _prep2(S1, wcat2)
    S2, s2b = _edge_layer(T2, edge_index, c2_b1, c2_g1, c2_be1,
                          c2_W2, c2_b2, c2_g2, c2_be2)

    l1_eff = jnp.concatenate(
        [s2a[:, None] * l1_W[:H], s2b[:, None] * l1_W[H:]], axis=0)
    r1, stA = _cat_mm(S1, S2, l1_eff, l1_b.reshape(1, -1))
    _, sA, cA = _bn_fold_n(stA, l1_g, l1_be)

    r2, stB = _node_mm(r1, sA[:, None] * m1_W,
                       (cA @ m1_W + m1_b).reshape(1, -1))
    _, sB, cB = _bn_fold_n(stB, m1_g, m1_be)

    r3, stC = _node_mm(r2, sB[:, None] * m2_W,
                       (cB @ m2_W + m2_b).reshape(1, -1))
    _, sC, cC = _bn_fold_n(stC, m2_g, m2_be)

    fw = sC[:, None] * f_W
    fb = cC @ f_W + f_b
    C = f_W.shape[1]
    CP = 16
    fw_pad = jnp.pad(fw, ((0, 0), (0, CP - C)))
    fb_pad = jnp.pad(fb, (0, CP - C), constant_values=-1e30)
    out = _final_mm(r3, fw_pad, fb_pad.reshape(1, CP))
    return out[:, :C]


# speedup vs baseline: 2.3466x; 2.3466x over previous
import functools

import jax
import jax.numpy as jnp
from jax import lax
from jax.experimental import pallas as pl
from jax.experimental.pallas import tpu as pltpu
from jax.experimental.pallas import tpu_sc as plsc

N = 10000
E = 320000
D = 128
H = 64
EPS = 1e-5

NC = 2
NS = 16
NW = NC * NS
CH = 128
EPW = E // NW
NFULL = EPW // CH
TAIL = EPW - NFULL * CH

_MESH = dict(core_axis_name="c", subcore_axis_name="s", num_cores=NC,
             num_subcores=NS)

ROWS_PER_SUB = N // NS


def _sc_gather(T, src, dst):
    @functools.partial(
        pl.kernel,
        out_type=jax.ShapeDtypeStruct((E, H), jnp.float32),
        mesh=plsc.VectorSubcoreMesh(**_MESH),
        scratch_types=[
            pltpu.VMEM((CH,), jnp.int32),
            pltpu.VMEM((CH,), jnp.int32),
            pltpu.VMEM((CH, H), jnp.float32),
            pltpu.VMEM((TAIL,), jnp.int32),
            pltpu.VMEM((TAIL,), jnp.int32),
            pltpu.VMEM((TAIL, H), jnp.float32),
        ],
        compiler_params=pltpu.CompilerParams(use_tc_tiling_on_sc=False),
    )
    def k(t_hbm, src_hbm, dst_hbm, pre_hbm, idxd, idxs, buf, idxd_t, idxs_t,
          buf_t):
        c = lax.axis_index("c")
        s = lax.axis_index("s")
        wid = s * NC + c

        def do_chunk(base, n, i_d, i_s, b):
            pltpu.sync_copy(dst_hbm.at[pl.ds(base, n)], i_d)
            pltpu.sync_copy(src_hbm.at[pl.ds(base, n)], i_s)
            for j in range(n // 16):
                sl = pl.ds(j * 16, 16)
                i_s[sl] = i_s[sl] + N
            pltpu.sync_copy(t_hbm.at[i_d], b)
            pltpu.sync_copy(t_hbm.at[i_s], b, add=True)
            pltpu.sync_copy(b, pre_hbm.at[pl.ds(base, n)])

        def body(i, carry):
            do_chunk(wid * EPW + i * CH, CH, idxd, idxs, buf)
            return carry

        lax.fori_loop(0, NFULL, body, 0)
        do_chunk(wid * EPW + NFULL * CH, TAIL, idxd_t, idxs_t, buf_t)

    return k(T, src, dst)


def _sc_scatter(h2, dst, kap):
    @functools.partial(
        pl.kernel,
        out_type=jax.ShapeDtypeStruct((NC, N, H), jnp.float32),
        mesh=plsc.VectorSubcoreMesh(**_MESH),
        scratch_types=[
            pltpu.VMEM_SHARED((N, H), jnp.float32),
            pltpu.VMEM((ROWS_PER_SUB, H), jnp.float32),
            pltpu.VMEM((CH,), jnp.int32),
            pltpu.VMEM((CH, H), jnp.float32),
            pltpu.VMEM((TAIL,), jnp.int32),
            pltpu.VMEM((TAIL, H), jnp.float32),
            pltpu.VMEM((H,), jnp.float32),
        ],
        compiler_params=pltpu.CompilerParams(use_tc_tiling_on_sc=False),
    )
    def k(h2_hbm, dst_hbm, kap_hbm, out_hbm, acc, stage, idx, buf, idx_t,
          buf_t, kapv):
        c = lax.axis_index("c")
        s = lax.axis_index("s")
        wid = s * NC + c

        def zrow(i, carry):
            for j in range(H // 16):
                stage[i, pl.ds(j * 16, 16)] = jnp.zeros((16,), jnp.float32)
            return carry

        lax.fori_loop(0, ROWS_PER_SUB, zrow, 0)
        pltpu.sync_copy(stage, acc.at[pl.ds(s * ROWS_PER_SUB, ROWS_PER_SUB)])
        pltpu.sync_copy(kap_hbm, kapv)
        plsc.subcore_barrier()

        def do_chunk(base, n, i_r, b):
            pltpu.sync_copy(dst_hbm.at[pl.ds(base, n)], i_r)
            pltpu.sync_copy(h2_hbm.at[pl.ds(base, n)], b)

            def row(r, carry):
                for j in range(H // 16):
                    sl = pl.ds(j * 16, 16)
                    b[r, sl] = b[r, sl] + kapv[sl]
                return carry

            lax.fori_loop(0, n, row, 0)
            pltpu.sync_copy(b, acc.at[i_r], add=True)

        def body(i, carry):
            do_chunk(wid * EPW + i * CH, CH, idx, buf)
            return carry

        lax.fori_loop(0, NFULL, body, 0)
        do_chunk(wid * EPW + NFULL * CH, TAIL, idx_t, buf_t)

        plsc.subcore_barrier()
        sl = pl.ds(s * ROWS_PER_SUB, ROWS_PER_SUB)
        pltpu.sync_copy(acc.at[sl], stage)
        pltpu.sync_copy(stage, out_hbm.at[c, sl])

    return k(h2, dst, kap)


def _prep1(x, wcat):
    BR, NB = 2000, N // 2000

    def body(x_ref, w_ref, o_ref):
        o_ref[...] = jnp.dot(x_ref[...], w_ref[0],
                             preferred_element_type=jnp.float32)

    return pl.pallas_call(
        body,
        grid=(2, NB),
        in_specs=[
            pl.BlockSpec((BR, D), lambda p, j: (j, 0)),
            pl.BlockSpec((1, D, H), lambda p, j: (p, 0, 0)),
        ],
        out_specs=pl.BlockSpec((BR, H), lambda p, j: (p * NB + j, 0)),
        out_shape=jax.ShapeDtypeStruct((2 * N, H), jnp.float32),
    )(x, wcat)


def _prep2(S, wcat):
    BR, NB = 2000, N // 2000

    def body(s_ref, w_ref, o_ref):
        xin = s_ref[0] + s_ref[1]
        o_ref[...] = jnp.dot(xin, w_ref[0],
                             preferred_element_type=jnp.float32)

    return pl.pallas_call(
        body,
        grid=(2, NB),
        in_specs=[
            pl.BlockSpec((2, BR, H), lambda p, j: (0, j, 0)),
            pl.BlockSpec((1, H, H), lambda p, j: (p, 0, 0)),
        ],
        out_specs=pl.BlockSpec((BR, H), lambda p, j: (p * NB + j, 0)),
        out_shape=jax.ShapeDtypeStruct((2 * N, H), jnp.float32),
    )(S, wcat)


def _edge_stats(pre, b1row):
    BR = 4000
    NB = E // BR

    def body(p_ref, b_ref, o_ref):
        h = jnp.maximum(p_ref[...] + b_ref[...], 0.0)

        @pl.when(pl.program_id(0) == 0)
        def _():
            o_ref[...] = jnp.zeros_like(o_ref)

        o_ref[0:1, :] += jnp.sum(h, axis=0, keepdims=True)
        o_ref[1:2, :] += jnp.sum(h * h, axis=0, keepdims=True)

    return pl.pallas_call(
        body,
        grid=(NB,),
        in_specs=[
            pl.BlockSpec((BR, H), lambda i: (i, 0)),
            pl.BlockSpec((1, H), lambda i: (0, 0)),
        ],
        out_specs=pl.BlockSpec((8, H), lambda i: (0, 0)),
        out_shape=jax.ShapeDtypeStruct((8, H), jnp.float32),
    )(pre, b1row)


def _edge_mm(pre, b1row, w2, b2row):
    BR = 4000
    NB = E // BR

    def body(p_ref, b1_ref, w_ref, b2_ref, h2_ref, st_ref):
        h1 = jnp.maximum(p_ref[...] + b1_ref[...], 0.0)
        h2 = jnp.maximum(
            jnp.dot(h1, w_ref[...], preferred_element_type=jnp.float32)
            + b2_ref[...], 0.0)
        h2_ref[...] = h2

        @pl.when(pl.program_id(0) == 0)
        def _():
            st_ref[...] = jnp.zeros_like(st_ref)

        st_ref[0:1, :] += jnp.sum(h2, axis=0, keepdims=True)
        st_ref[1:2, :] += jnp.sum(h2 * h2, axis=0, keepdims=True)

    return pl.pallas_call(
        body,
        grid=(NB,),
        in_specs=[
            pl.BlockSpec((BR, H), lambda i: (i, 0)),
            pl.BlockSpec((1, H), lambda i: (0, 0)),
            pl.BlockSpec((H, H), lambda i: (0, 0)),
            pl.BlockSpec((1, H), lambda i: (0, 0)),
        ],
        out_specs=[
            pl.BlockSpec((BR, H), lambda i: (i, 0)),
            pl.BlockSpec((8, H), lambda i: (0, 0)),
        ],
        out_shape=[
            jax.ShapeDtypeStruct((E, H), jnp.float32),
            jax.ShapeDtypeStruct((8, H), jnp.float32),
        ],
    )(pre, b1row, w2, b2row)


def _cat_mm(S1, S2, w, brow):
    BR, NB = 2000, N // 2000
    MH = w.shape[1]

    def body(s1_ref, s2_ref, w_ref, b_ref, r_ref, st_ref):
        x1 = s1_ref[0] + s1_ref[1]
        x2 = s2_ref[0] + s2_ref[1]
        xcat = jnp.concatenate([x1, x2], axis=1)
        r = jnp.maximum(
            jnp.dot(xcat, w_ref[...], preferred_element_type=jnp.float32)
            + b_ref[...], 0.0)
        r_ref[...] = r

        @pl.when(pl.program_id(0) == 0)
        def _():
            st_ref[...] = jnp.zeros_like(st_ref)

        st_ref[0:1, :] += jnp.sum(r, axis=0, keepdims=True)
        st_ref[1:2, :] += jnp.sum(r * r, axis=0, keepdims=True)

    return pl.pallas_call(
        body,
        grid=(NB,),
        in_specs=[
            pl.BlockSpec((2, BR, H), lambda j: (0, j, 0)),
            pl.BlockSpec((2, BR, H), lambda j: (0, j, 0)),
            pl.BlockSpec((2 * H, MH), lambda j: (0, 0)),
            pl.BlockSpec((1, MH), lambda j: (0, 0)),
        ],
        out_specs=[
            pl.BlockSpec((BR, MH), lambda j: (j, 0)),
            pl.BlockSpec((8, MH), lambda j: (0, 0)),
        ],
        out_shape=[
            jax.ShapeDtypeStruct((N, MH), jnp.float32),
            jax.ShapeDtypeStruct((8, MH), jnp.float32),
        ],
    )(S1, S2, w, brow)


def _node_mm(xin, w, brow):
    BR, NB = 2000, N // 2000
    K, M = w.shape

    def body(x_ref, w_ref, b_ref, r_ref, st_ref):
        r = jnp.maximum(
            jnp.dot(x_ref[...], w_ref[...],
                    preferred_element_type=jnp.float32) + b_ref[...], 0.0)
        r_ref[...] = r

        @pl.when(pl.program_id(0) == 0)
        def _():
            st_ref[...] = jnp.zeros_like(st_ref)

        st_ref[0:1, :] += jnp.sum(r, axis=0, keepdims=True)
        st_ref[1:2, :] += jnp.sum(r * r, axis=0, keepdims=True)

    return pl.pallas_call(
        body,
        grid=(NB,),
        in_specs=[
            pl.BlockSpec((BR, K), lambda j: (j, 0)),
            pl.BlockSpec((K, M), lambda j: (0, 0)),
            pl.BlockSpec((1, M), lambda j: (0, 0)),
        ],
        out_specs=[
            pl.BlockSpec((BR, M), lambda j: (j, 0)),
            pl.BlockSpec((8, M), lambda j: (0, 0)),
        ],
        out_shape=[
            jax.ShapeDtypeStruct((N, M), jnp.float32),
            jax.ShapeDtypeStruct((8, M), jnp.float32),
        ],
    )(xin, w, brow)


def _final_mm(xin, w, brow):
    BR, NB = 2000, N // 2000
    K, M = w.shape

    def body(x_ref, w_ref, b_ref, o_ref):
        z = jnp.dot(x_ref[...], w_ref[...],
                    preferred_element_type=jnp.float32) + b_ref[...]
        m = jnp.max(z, axis=1, keepdims=True)
        lse = jnp.log(jnp.sum(jnp.exp(z - m), axis=1, keepdims=True)) + m
        o_ref[...] = z - lse

    return pl.pallas_call(
        body,
        grid=(NB,),
        in_specs=[
            pl.BlockSpec((BR, K), lambda j: (j, 0)),
            pl.BlockSpec((K, M), lambda j: (0, 0)),
            pl.BlockSpec((1, M), lambda j: (0, 0)),
        ],
        out_specs=pl.BlockSpec((BR, M), lambda j: (j, 0)),
        out_shape=jax.ShapeDtypeStruct((N, M), jnp.float32),
    )(xin, w, brow)


def _bn_fold(st, g, be):
    mu = st[0] / E
    var = st[1] / E - mu * mu
    s = g / jnp.sqrt(var + EPS)
    return mu, s, be - s * mu


def _bn_fold_n(st, g, be):
    mu = st[0] / N
    var = st[1] / N - mu * mu
    s = g / jnp.sqrt(var + EPS)
    return mu, s, be - s * mu


def _edge_layer(xin_T, src, dst, b1, g1, be1, W2, b2, g2, be2):
    pre = _sc_gather(xin_T, src, dst)
    st1 = _edge_stats(pre, b1.reshape(1, H))
    _, s1, c1 = _bn_fold(st1, g1, be1)
    w2p = s1[:, None] * W2
    b2p = c1 @ W2 + b2
    h2, st2 = _edge_mm(pre, b1.reshape(1, H), w2p, b2p.reshape(1, H))
    mu2 = st2[0] / E
    var2 = st2[1] / E - mu2 * mu2
    s2 = g2 / jnp.sqrt(var2 + EPS)
    kap = be2 / s2 - mu2
    S = _sc_scatter(h2, dst, kap)
    return S, s2


def kernel(x, edge_index, c1_W1, c1_b1, c1_g1, c1_be1, c1_W2, c1_b2, c1_g2,
           c1_be2, c2_W1, c2_b1, c2_g1, c2_be1, c2_W2, c2_b2, c2_g2, c2_be2,
           l1_W, l1_b, l1_g, l1_be, m1_W, m1_b, m1_g, m1_be, m2_W, m2_b,
           m2_g, m2_be, f_W, f_b):
    src = edge_index[0]
    dst = edge_index[1]

    wcat1 = jnp.stack([c1_W1[:D] - c1_W1[D:], c1_W1[D:]])
    T1 = _prep1(x, wcat1)
    S1, s2a = _edge_layer(T1, src, dst, c1_b1, c1_g1, c1_be1,
                          c1_W2, c1_b2, c1_g2, c1_be2)

    wcat2 = s2a[None, :, None] * jnp.stack(
        [c2_W1[:H] - c2_W1[H:], c2_W1[H:]])
    T2 = _prep2(S1, wcat2)
    S2, s2b = _edge_layer(T2, src, dst, c2_b1, c2_g1, c2_be1,
                          c2_W2, c2_b2, c2_g2, c2_be2)

    l1_eff = jnp.concatenate(
        [s2a[:, None] * l1_W[:H], s2b[:, None] * l1_W[H:]], axis=0)
    r1, stA = _cat_mm(S1, S2, l1_eff, l1_b.reshape(1, -1))
    _, sA, cA = _bn_fold_n(stA, l1_g, l1_be)

    r2, stB = _node_mm(r1, sA[:, None] * m1_W,
                       (cA @ m1_W + m1_b).reshape(1, -1))
    _, sB, cB = _bn_fold_n(stB, m1_g, m1_be)

    r3, stC = _node_mm(r2, sB[:, None] * m2_W,
                       (cB @ m2_W + m2_b).reshape(1, -1))
    _, sC, cC = _bn_fold_n(stC, m2_g, m2_be)

    fw = sC[:, None] * f_W
    fb = cC @ f_W + f_b
    C = f_W.shape[1]
    CP = 16
    fw_pad = jnp.pad(fw, ((0, 0), (0, CP - C)))
    fb_pad = jnp.pad(fb, (0, CP - C), constant_values=-1e30)
    out = _final_mm(r3, fw_pad, fb_pad.reshape(1, CP))
    return out[:, :C]


# speedup vs baseline: 3.4182x; 1.4567x over previous
import functools

import jax
import jax.numpy as jnp
from jax import lax
from jax.experimental import pallas as pl
from jax.experimental.pallas import tpu as pltpu
from jax.experimental.pallas import tpu_sc as plsc

N = 10000
E = 320000
D = 128
H = 64
EPS = 1e-5

NC = 2
NS = 16
NW = NC * NS
CH = 128
EPW = E // NW
NFULL = EPW // CH
TAIL = EPW - NFULL * CH

_MESH = dict(core_axis_name="c", subcore_axis_name="s", num_cores=NC,
             num_subcores=NS)

ROWS_PER_SUB = N // NS


def _sc_gather(T, src, dst):
    @functools.partial(
        pl.kernel,
        out_type=jax.ShapeDtypeStruct((E, H), jnp.float32),
        mesh=plsc.VectorSubcoreMesh(**_MESH),
        scratch_types=[
            pltpu.VMEM((CH,), jnp.int32),
            pltpu.VMEM((CH,), jnp.int32),
            pltpu.VMEM((CH, H), jnp.float32),
            pltpu.VMEM((TAIL,), jnp.int32),
            pltpu.VMEM((TAIL,), jnp.int32),
            pltpu.VMEM((TAIL, H), jnp.float32),
        ],
        compiler_params=pltpu.CompilerParams(use_tc_tiling_on_sc=False),
    )
    def k(t_hbm, src_hbm, dst_hbm, pre_hbm, idxd, idxs, buf, idxd_t, idxs_t,
          buf_t):
        c = lax.axis_index("c")
        s = lax.axis_index("s")
        wid = s * NC + c

        def do_chunk(base, n, i_d, i_s, b):
            pltpu.sync_copy(dst_hbm.at[pl.ds(base, n)], i_d)
            pltpu.sync_copy(src_hbm.at[pl.ds(base, n)], i_s)
            for j in range(n // 16):
                sl = pl.ds(j * 16, 16)
                i_s[sl] = i_s[sl] + N
            pltpu.sync_copy(t_hbm.at[i_d], b)
            pltpu.sync_copy(t_hbm.at[i_s], b, add=True)
            pltpu.sync_copy(b, pre_hbm.at[pl.ds(base, n)])

        def body(i, carry):
            do_chunk(wid * EPW + i * CH, CH, idxd, idxs, buf)
            return carry

        lax.fori_loop(0, NFULL, body, 0)
        do_chunk(wid * EPW + NFULL * CH, TAIL, idxd_t, idxs_t, buf_t)

    return k(T, src, dst)


def _sc_scatter(h2, dst, kap):
    @functools.partial(
        pl.kernel,
        out_type=jax.ShapeDtypeStruct((NC, N, H), jnp.float32),
        mesh=plsc.VectorSubcoreMesh(**_MESH),
        scratch_types=[
            pltpu.VMEM_SHARED((N, H), jnp.float32),
            pltpu.VMEM((ROWS_PER_SUB, H), jnp.float32),
            pltpu.VMEM((CH,), jnp.int32),
            pltpu.VMEM((CH, H), jnp.float32),
            pltpu.VMEM((TAIL,), jnp.int32),
            pltpu.VMEM((TAIL, H), jnp.float32),
            pltpu.VMEM((H,), jnp.float32),
        ],
        compiler_params=pltpu.CompilerParams(use_tc_tiling_on_sc=False),
    )
    def k(h2_hbm, dst_hbm, kap_hbm, out_hbm, acc, stage, idx, buf, idx_t,
          buf_t, kapv):
        c = lax.axis_index("c")
        s = lax.axis_index("s")
        wid = s * NC + c

        def zrow(i, carry):
            for j in range(H // 16):
                stage[i, pl.ds(j * 16, 16)] = jnp.zeros((16,), jnp.float32)
            return carry

        lax.fori_loop(0, ROWS_PER_SUB, zrow, 0)
        pltpu.sync_copy(stage, acc.at[pl.ds(s * ROWS_PER_SUB, ROWS_PER_SUB)])
        pltpu.sync_copy(kap_hbm, kapv)
        plsc.subcore_barrier()

        def do_chunk(base, n, i_r, b):
            pltpu.sync_copy(dst_hbm.at[pl.ds(base, n)], i_r)
            pltpu.sync_copy(h2_hbm.at[pl.ds(base, n)], b)

            def row(r, carry):
                for j in range(H // 16):
                    sl = pl.ds(j * 16, 16)
                    b[r, sl] = b[r, sl] + kapv[sl]
                return carry

            lax.fori_loop(0, n, row, 0)
            pltpu.sync_copy(b, acc.at[i_r], add=True)

        def body(i, carry):
            do_chunk(wid * EPW + i * CH, CH, idx, buf)
            return carry

        lax.fori_loop(0, NFULL, body, 0)
        do_chunk(wid * EPW + NFULL * CH, TAIL, idx_t, buf_t)

        plsc.subcore_barrier()
        sl = pl.ds(s * ROWS_PER_SUB, ROWS_PER_SUB)
        pltpu.sync_copy(acc.at[sl], stage)
        pltpu.sync_copy(stage, out_hbm.at[c, sl])

    return k(h2, dst, kap)


def _prep1(x, wcat):
    BR, NB = 2000, N // 2000

    def body(x_ref, w_ref, o_ref):
        o_ref[...] = jnp.dot(x_ref[...], w_ref[0],
                             preferred_element_type=jnp.float32)

    return pl.pallas_call(
        body,
        grid=(2, NB),
        in_specs=[
            pl.BlockSpec((BR, D), lambda p, j: (j, 0)),
            pl.BlockSpec((1, D, H), lambda p, j: (p, 0, 0)),
        ],
        out_specs=pl.BlockSpec((BR, H), lambda p, j: (p * NB + j, 0)),
        out_shape=jax.ShapeDtypeStruct((2 * N, H), jnp.float32),
    )(x, wcat)


def _prep2(S, wcat):
    BR, NB = 2000, N // 2000

    def body(s_ref, w_ref, o_ref):
        xin = s_ref[0] + s_ref[1]
        o_ref[...] = jnp.dot(xin, w_ref[0],
                             preferred_element_type=jnp.float32)

    return pl.pallas_call(
        body,
        grid=(2, NB),
        in_specs=[
            pl.BlockSpec((2, BR, H), lambda p, j: (0, j, 0)),
            pl.BlockSpec((1, H, H), lambda p, j: (p, 0, 0)),
        ],
        out_specs=pl.BlockSpec((BR, H), lambda p, j: (p * NB + j, 0)),
        out_shape=jax.ShapeDtypeStruct((2 * N, H), jnp.float32),
    )(S, wcat)


def _edge_stats(pre128, b1row):
    BR = 4000
    E2 = E // 2
    NB = E2 // BR

    def body(p_ref, b_ref, o_ref):
        h = jnp.maximum(p_ref[...] + b_ref[...], 0.0)

        @pl.when(pl.program_id(0) == 0)
        def _():
            o_ref[...] = jnp.zeros_like(o_ref)

        o_ref[0:1, :] += jnp.sum(h, axis=0, keepdims=True)
        o_ref[1:2, :] += jnp.sum(h * h, axis=0, keepdims=True)

    return pl.pallas_call(
        body,
        grid=(NB,),
        in_specs=[
            pl.BlockSpec((BR, 2 * H), lambda i: (i, 0)),
            pl.BlockSpec((1, 2 * H), lambda i: (0, 0)),
        ],
        out_specs=pl.BlockSpec((8, 2 * H), lambda i: (0, 0)),
        out_shape=jax.ShapeDtypeStruct((8, 2 * H), jnp.float32),
    )(pre128, b1row)


def _edge_mm(pre128, b1row, w2bd, b2row):
    BR = 4000
    E2 = E // 2
    NB = E2 // BR

    def body(p_ref, b1_ref, w_ref, b2_ref, h2_ref, st_ref):
        h1 = jnp.maximum(p_ref[...] + b1_ref[...], 0.0)
        h2 = jnp.maximum(
            jnp.dot(h1, w_ref[...], preferred_element_type=jnp.float32)
            + b2_ref[...], 0.0)
        h2_ref[...] = h2

        @pl.when(pl.program_id(0) == 0)
        def _():
            st_ref[...] = jnp.zeros_like(st_ref)

        st_ref[0:1, :] += jnp.sum(h2, axis=0, keepdims=True)
        st_ref[1:2, :] += jnp.sum(h2 * h2, axis=0, keepdims=True)

    return pl.pallas_call(
        body,
        grid=(NB,),
        in_specs=[
            pl.BlockSpec((BR, 2 * H), lambda i: (i, 0)),
            pl.BlockSpec((1, 2 * H), lambda i: (0, 0)),
            pl.BlockSpec((2 * H, 2 * H), lambda i: (0, 0)),
            pl.BlockSpec((1, 2 * H), lambda i: (0, 0)),
        ],
        out_specs=[
            pl.BlockSpec((BR, 2 * H), lambda i: (i, 0)),
            pl.BlockSpec((8, 2 * H), lambda i: (0, 0)),
        ],
        out_shape=[
            jax.ShapeDtypeStruct((E2, 2 * H), jnp.float32),
            jax.ShapeDtypeStruct((8, 2 * H), jnp.float32),
        ],
    )(pre128, b1row, w2bd, b2row)


def _cat_mm(S1, S2, w, brow):
    BR, NB = 2000, N // 2000
    MH = w.shape[1]

    def body(s1_ref, s2_ref, w_ref, b_ref, r_ref, st_ref):
        x1 = s1_ref[0] + s1_ref[1]
        x2 = s2_ref[0] + s2_ref[1]
        xcat = jnp.concatenate([x1, x2], axis=1)
        r = jnp.maximum(
            jnp.dot(xcat, w_ref[...], preferred_element_type=jnp.float32)
            + b_ref[...], 0.0)
        r_ref[...] = r

        @pl.when(pl.program_id(0) == 0)
        def _():
            st_ref[...] = jnp.zeros_like(st_ref)

        st_ref[0:1, :] += jnp.sum(r, axis=0, keepdims=True)
        st_ref[1:2, :] += jnp.sum(r * r, axis=0, keepdims=True)

    return pl.pallas_call(
        body,
        grid=(NB,),
        in_specs=[
            pl.BlockSpec((2, BR, H), lambda j: (0, j, 0)),
            pl.BlockSpec((2, BR, H), lambda j: (0, j, 0)),
            pl.BlockSpec((2 * H, MH), lambda j: (0, 0)),
            pl.BlockSpec((1, MH), lambda j: (0, 0)),
        ],
        out_specs=[
            pl.BlockSpec((BR, MH), lambda j: (j, 0)),
            pl.BlockSpec((8, MH), lambda j: (0, 0)),
        ],
        out_shape=[
            jax.ShapeDtypeStruct((N, MH), jnp.float32),
            jax.ShapeDtypeStruct((8, MH), jnp.float32),
        ],
    )(S1, S2, w, brow)


def _node_mm(xin, w, brow):
    BR, NB = 2000, N // 2000
    K, M = w.shape

    def body(x_ref, w_ref, b_ref, r_ref, st_ref):
        r = jnp.maximum(
            jnp.dot(x_ref[...], w_ref[...],
                    preferred_element_type=jnp.float32) + b_ref[...], 0.0)
        r_ref[...] = r

        @pl.when(pl.program_id(0) == 0)
        def _():
            st_ref[...] = jnp.zeros_like(st_ref)

        st_ref[0:1, :] += jnp.sum(r, axis=0, keepdims=True)
        st_ref[1:2, :] += jnp.sum(r * r, axis=0, keepdims=True)

    return pl.pallas_call(
        body,
        grid=(NB,),
        in_specs=[
            pl.BlockSpec((BR, K), lambda j: (j, 0)),
            pl.BlockSpec((K, M), lambda j: (0, 0)),
            pl.BlockSpec((1, M), lambda j: (0, 0)),
        ],
        out_specs=[
            pl.BlockSpec((BR, M), lambda j: (j, 0)),
            pl.BlockSpec((8, M), lambda j: (0, 0)),
        ],
        out_shape=[
            jax.ShapeDtypeStruct((N, M), jnp.float32),
            jax.ShapeDtypeStruct((8, M), jnp.float32),
        ],
    )(xin, w, brow)


def _final_mm(xin, w, brow):
    BR, NB = 2000, N // 2000
    K, M = w.shape

    def body(x_ref, w_ref, b_ref, o_ref):
        z = jnp.dot(x_ref[...], w_ref[...],
                    preferred_element_type=jnp.float32) + b_ref[...]
        m = jnp.max(z, axis=1, keepdims=True)
        lse = jnp.log(jnp.sum(jnp.exp(z - m), axis=1, keepdims=True)) + m
        o_ref[...] = z - lse

    return pl.pallas_call(
        body,
        grid=(NB,),
        in_specs=[
            pl.BlockSpec((BR, K), lambda j: (j, 0)),
            pl.BlockSpec((K, M), lambda j: (0, 0)),
            pl.BlockSpec((1, M), lambda j: (0, 0)),
        ],
        out_specs=pl.BlockSpec((BR, M), lambda j: (j, 0)),
        out_shape=jax.ShapeDtypeStruct((N, M), jnp.float32),
    )(xin, w, brow)


def _bn_fold(st, g, be):
    mu = st[0] / E
    var = st[1] / E - mu * mu
    s = g / jnp.sqrt(var + EPS)
    return mu, s, be - s * mu


def _bn_fold_n(st, g, be):
    mu = st[0] / N
    var = st[1] / N - mu * mu
    s = g / jnp.sqrt(var + EPS)
    return mu, s, be - s * mu


def _edge_layer(xin_T, src, dst, b1, g1, be1, W2, b2, g2, be2):
    pre = _sc_gather(xin_T, src, dst)
    pre128 = pre.reshape(E // 2, 2 * H)
    b1r = jnp.tile(b1, 2).reshape(1, 2 * H)
    st1p = _edge_stats(pre128, b1r)
    st1 = st1p[:, :H] + st1p[:, H:]
    _, s1, c1 = _bn_fold(st1, g1, be1)
    w2p = s1[:, None] * W2
    b2p = c1 @ W2 + b2
    w2bd = jnp.zeros((2 * H, 2 * H), jnp.float32)
    w2bd = w2bd.at[:H, :H].set(w2p).at[H:, H:].set(w2p)
    h2_128, st2p = _edge_mm(pre128, b1r, w2bd, jnp.tile(b2p, 2).reshape(1, -1))
    st2 = st2p[:, :H] + st2p[:, H:]
    mu2 = st2[0] / E
    var2 = st2[1] / E - mu2 * mu2
    s2 = g2 / jnp.sqrt(var2 + EPS)
    kap = be2 / s2 - mu2
    S = _sc_scatter(h2_128.reshape(E, H), dst, kap)
    return S, s2


def kernel(x, edge_index, c1_W1, c1_b1, c1_g1, c1_be1, c1_W2, c1_b2, c1_g2,
           c1_be2, c2_W1, c2_b1, c2_g1, c2_be1, c2_W2, c2_b2, c2_g2, c2_be2,
           l1_W, l1_b, l1_g, l1_be, m1_W, m1_b, m1_g, m1_be, m2_W, m2_b,
           m2_g, m2_be, f_W, f_b):
    src = edge_index[0]
    dst = edge_index[1]

    wcat1 = jnp.stack([c1_W1[:D] - c1_W1[D:], c1_W1[D:]])
    T1 = _prep1(x, wcat1)
    S1, s2a = _edge_layer(T1, src, dst, c1_b1, c1_g1, c1_be1,
                          c1_W2, c1_b2, c1_g2, c1_be2)

    wcat2 = s2a[None, :, None] * jnp.stack(
        [c2_W1[:H] - c2_W1[H:], c2_W1[H:]])
    T2 = _prep2(S1, wcat2)
    S2, s2b = _edge_layer(T2, src, dst, c2_b1, c2_g1, c2_be1,
                          c2_W2, c2_b2, c2_g2, c2_be2)

    l1_eff = jnp.concatenate(
        [s2a[:, None] * l1_W[:H], s2b[:, None] * l1_W[H:]], axis=0)
    r1, stA = _cat_mm(S1, S2, l1_eff, l1_b.reshape(1, -1))
    _, sA, cA = _bn_fold_n(stA, l1_g, l1_be)

    r2, stB = _node_mm(r1, sA[:, None] * m1_W,
                       (cA @ m1_W + m1_b).reshape(1, -1))
    _, sB, cB = _bn_fold_n(stB, m1_g, m1_be)

    r3, stC = _node_mm(r2, sB[:, None] * m2_W,
                       (cB @ m2_W + m2_b).reshape(1, -1))
    _, sC, cC = _bn_fold_n(stC, m2_g, m2_be)

    fw = sC[:, None] * f_W
    fb = cC @ f_W + f_b
    C = f_W.shape[1]
    CP = 16
    fw_pad = jnp.pad(fw, ((0, 0), (0, CP - C)))
    fb_pad = jnp.pad(fb, (0, CP - C), constant_values=-1e30)
    out = _final_mm(r3, fw_pad, fb_pad.reshape(1, CP))
    return out[:, :C]


# speedup vs baseline: 5.3512x; 1.5655x over previous
import functools

import jax
import jax.numpy as jnp
from jax import lax
from jax.experimental import pallas as pl
from jax.experimental.pallas import tpu as pltpu
from jax.experimental.pallas import tpu_sc as plsc

N = 10000
E = 320000
D = 128
H = 64
EPS = 1e-5

NC = 2
NS = 16
NW = NC * NS
CH = 128
EPW = E // NW
NFULL = EPW // CH
TAIL = EPW - NFULL * CH

_MESH = dict(core_axis_name="c", subcore_axis_name="s", num_cores=NC,
             num_subcores=NS)

ROWS_PER_SUB = N // NS


def _sc_gather(T, src, dst):
    @functools.partial(
        pl.kernel,
        out_type=jax.ShapeDtypeStruct((E, H), jnp.float32),
        mesh=plsc.VectorSubcoreMesh(**_MESH),
        scratch_types=[
            pltpu.VMEM((EPW,), jnp.int32),
            pltpu.VMEM((EPW,), jnp.int32),
            pltpu.VMEM((CH, H), jnp.float32),
            pltpu.VMEM((CH, H), jnp.float32),
            pltpu.VMEM((TAIL, H), jnp.float32),
            pltpu.SemaphoreType.DMA,
            pltpu.SemaphoreType.DMA,
        ],
        compiler_params=pltpu.CompilerParams(use_tc_tiling_on_sc=False),
    )
    def k(t_hbm, src_hbm, dst_hbm, pre_hbm, idxd, idxs, buf0, buf1, buf_t,
          sem0, sem1):
        c = lax.axis_index("c")
        s = lax.axis_index("s")
        wid = s * NC + c
        base0 = wid * EPW

        pltpu.sync_copy(dst_hbm.at[pl.ds(base0, EPW)], idxd)
        pltpu.sync_copy(src_hbm.at[pl.ds(base0, EPW)], idxs)

        def shift(i, carry):
            sl = pl.ds(i * 16, 16)
            idxs[sl] = idxs[sl] + N
            return carry

        lax.fori_loop(0, EPW // 16, shift, 0)

        def chunk(ci, b, sem, first):
            base = base0 + ci * CH

            @pl.when(jnp.logical_not(first))
            def _():
                pltpu.make_async_copy(b, pre_hbm.at[pl.ds(base, CH)],
                                      sem).wait()

            pltpu.sync_copy(t_hbm.at[idxd.at[pl.ds(ci * CH, CH)]], b)
            pltpu.sync_copy(t_hbm.at[idxs.at[pl.ds(ci * CH, CH)]], b,
                            add=True)
            pltpu.async_copy(b, pre_hbm.at[pl.ds(base, CH)], sem)

        def body(i, carry):
            chunk(2 * i, buf0, sem0, i == 0)
            chunk(2 * i + 1, buf1, sem1, i == 0)
            return carry

        lax.fori_loop(0, NFULL // 2, body, 0)
        pltpu.make_async_copy(buf0, pre_hbm.at[pl.ds(base0, CH)], sem0).wait()
        pltpu.make_async_copy(buf1, pre_hbm.at[pl.ds(base0, CH)], sem1).wait()

        tb = NFULL * CH
        pltpu.sync_copy(t_hbm.at[idxd.at[pl.ds(tb, TAIL)]], buf_t)
        pltpu.sync_copy(t_hbm.at[idxs.at[pl.ds(tb, TAIL)]], buf_t, add=True)
        pltpu.sync_copy(buf_t, pre_hbm.at[pl.ds(base0 + tb, TAIL)])

    return k(T, src, dst)


DW = 16


def _sc_scatter(h2, dst, with_deg):
    out_type = [jax.ShapeDtypeStruct((NC, N, H), jnp.float32)]
    scratch = [
        pltpu.VMEM_SHARED((N, H), jnp.float32),
        pltpu.VMEM((ROWS_PER_SUB, H), jnp.float32),
        pltpu.VMEM((CH,), jnp.int32),
        pltpu.VMEM((CH,), jnp.int32),
        pltpu.VMEM((CH, H), jnp.float32),
        pltpu.VMEM((CH, H), jnp.float32),
        pltpu.VMEM((TAIL,), jnp.int32),
        pltpu.VMEM((TAIL, H), jnp.float32),
        pltpu.SemaphoreType.DMA,
        pltpu.SemaphoreType.DMA,
    ]
    if with_deg:
        out_type.append(jax.ShapeDtypeStruct((NC, N, DW), jnp.float32))
        scratch += [
            pltpu.VMEM_SHARED((N, DW), jnp.float32),
            pltpu.VMEM((ROWS_PER_SUB, DW), jnp.float32),
            pltpu.VMEM((CH, DW), jnp.float32),
            pltpu.VMEM((TAIL, DW), jnp.float32),
        ]

    @functools.partial(
        pl.kernel,
        out_type=tuple(out_type),
        mesh=plsc.VectorSubcoreMesh(**_MESH),
        scratch_types=scratch,
        compiler_params=pltpu.CompilerParams(use_tc_tiling_on_sc=False),
    )
    def k(h2_hbm, dst_hbm, out_hbm, *rest):
        if with_deg:
            (outd_hbm, acc, stage, idx0, idx1, buf0, buf1, idx_t, buf_t,
             sem0, sem1, dacc, dstage, ones, ones_t) = rest
        else:
            (acc, stage, idx0, idx1, buf0, buf1, idx_t, buf_t,
             sem0, sem1) = rest
        c = lax.axis_index("c")
        s = lax.axis_index("s")
        wid = s * NC + c
        base0 = wid * EPW
        rsl = pl.ds(s * ROWS_PER_SUB, ROWS_PER_SUB)

        def zrow(i, carry):
            for j in range(H // 16):
                stage[i, pl.ds(j * 16, 16)] = jnp.zeros((16,), jnp.float32)
            return carry

        lax.fori_loop(0, ROWS_PER_SUB, zrow, 0)
        pltpu.sync_copy(stage, acc.at[rsl])
        if with_deg:
            def zdrow(i, carry):
                dstage[i, pl.ds(0, 16)] = jnp.zeros((16,), jnp.float32)
                return carry

            lax.fori_loop(0, ROWS_PER_SUB, zdrow, 0)
            pltpu.sync_copy(dstage, dacc.at[rsl])

            def orow(i, carry):
                ones[i, pl.ds(0, 16)] = jnp.ones((16,), jnp.float32)
                return carry

            lax.fori_loop(0, CH, orow, 0)
            for i in range(TAIL):
                ones_t[i, pl.ds(0, 16)] = jnp.ones((16,), jnp.float32)
        plsc.subcore_barrier()

        def chunk(ci, i_r, b, sem, first):
            base = base0 + ci * CH

            @pl.when(jnp.logical_not(first))
            def _():
                pltpu.make_async_copy(b, acc.at[i_r], sem).wait()

            pltpu.sync_copy(dst_hbm.at[pl.ds(base, CH)], i_r)
            pltpu.sync_copy(h2_hbm.at[pl.ds(base, CH)], b)
            if with_deg:
                pltpu.sync_copy(ones, dacc.at[i_r], add=True)
            pltpu.async_copy(b, acc.at[i_r], sem, add=True)

        def body(i, carry):
            chunk(2 * i, idx0, buf0, sem0, i == 0)
            chunk(2 * i + 1, idx1, buf1, sem1, i == 0)
            return carry

        lax.fori_loop(0, NFULL // 2, body, 0)
        pltpu.make_async_copy(buf0, acc.at[idx0], sem0).wait()
        pltpu.make_async_copy(buf1, acc.at[idx1], sem1).wait()

        tb = base0 + NFULL * CH
        pltpu.sync_copy(dst_hbm.at[pl.ds(tb, TAIL)], idx_t)
        pltpu.sync_copy(h2_hbm.at[pl.ds(tb, TAIL)], buf_t)
        pltpu.sync_copy(buf_t, acc.at[idx_t], add=True)
        if with_deg:
            pltpu.sync_copy(ones_t, dacc.at[idx_t], add=True)

        plsc.subcore_barrier()
        pltpu.sync_copy(acc.at[rsl], stage)
        pltpu.sync_copy(stage, out_hbm.at[c, rsl])
        if with_deg:
            pltpu.sync_copy(dacc.at[rsl], dstage)
            pltpu.sync_copy(dstage, outd_hbm.at[c, rsl])

    res = k(h2, dst)
    return res if with_deg else res[0]


def _prep1(x, wcat):
    BR, NB = 2000, N // 2000

    def body(x_ref, w_ref, o_ref):
        o_ref[...] = jnp.dot(x_ref[...], w_ref[0],
                             preferred_element_type=jnp.float32)

    return pl.pallas_call(
        body,
        grid=(2, NB),
        in_specs=[
            pl.BlockSpec((BR, D), lambda p, j: (j, 0)),
            pl.BlockSpec((1, D, H), lambda p, j: (p, 0, 0)),
        ],
        out_specs=pl.BlockSpec((BR, H), lambda p, j: (p * NB + j, 0)),
        out_shape=jax.ShapeDtypeStruct((2 * N, H), jnp.float32),
    )(x, wcat)


def _prep2(S, wcat, deg, vp):
    BR, NB = 2000, N // 2000

    def body(s_ref, w_ref, d_ref, v_ref, o_ref):
        xin = s_ref[0] + s_ref[1]
        o_ref[...] = (jnp.dot(xin, w_ref[0],
                              preferred_element_type=jnp.float32)
                      + d_ref[...] * v_ref[0])

    return pl.pallas_call(
        body,
        grid=(2, NB),
        in_specs=[
            pl.BlockSpec((2, BR, H), lambda p, j: (0, j, 0)),
            pl.BlockSpec((1, H, H), lambda p, j: (p, 0, 0)),
            pl.BlockSpec((BR, 1), lambda p, j: (j, 0)),
            pl.BlockSpec((1, 1, H), lambda p, j: (p, 0, 0)),
        ],
        out_specs=pl.BlockSpec((BR, H), lambda p, j: (p * NB + j, 0)),
        out_shape=jax.ShapeDtypeStruct((2 * N, H), jnp.float32),
    )(S, wcat, deg, vp)


def _edge_stats(pre128, b1row):
    BR = 4000
    E2 = E // 2
    NB = E2 // BR

    def body(p_ref, b_ref, o_ref):
        h = jnp.maximum(p_ref[...] + b_ref[...], 0.0)

        @pl.when(pl.program_id(0) == 0)
        def _():
            o_ref[...] = jnp.zeros_like(o_ref)

        o_ref[0:1, :] += jnp.sum(h, axis=0, keepdims=True)
        o_ref[1:2, :] += jnp.sum(h * h, axis=0, keepdims=True)

    return pl.pallas_call(
        body,
        grid=(NB,),
        in_specs=[
            pl.BlockSpec((BR, 2 * H), lambda i: (i, 0)),
            pl.BlockSpec((1, 2 * H), lambda i: (0, 0)),
        ],
        out_specs=pl.BlockSpec((8, 2 * H), lambda i: (0, 0)),
        out_shape=jax.ShapeDtypeStruct((8, 2 * H), jnp.float32),
    )(pre128, b1row)


def _edge_mm(pre128, b1row, w2bd, b2row):
    BR = 4000
    E2 = E // 2
    NB = E2 // BR

    def body(p_ref, b1_ref, w_ref, b2_ref, h2_ref, st_ref):
        h1 = jnp.maximum(p_ref[...] + b1_ref[...], 0.0)
        h2 = jnp.maximum(
            jnp.dot(h1, w_ref[...], preferred_element_type=jnp.float32)
            + b2_ref[...], 0.0)
        h2_ref[...] = h2

        @pl.when(pl.program_id(0) == 0)
        def _():
            st_ref[...] = jnp.zeros_like(st_ref)

        st_ref[0:1, :] += jnp.sum(h2, axis=0, keepdims=True)
        st_ref[1:2, :] += jnp.sum(h2 * h2, axis=0, keepdims=True)

    return pl.pallas_call(
        body,
        grid=(NB,),
        in_specs=[
            pl.BlockSpec((BR, 2 * H), lambda i: (i, 0)),
            pl.BlockSpec((1, 2 * H), lambda i: (0, 0)),
            pl.BlockSpec((2 * H, 2 * H), lambda i: (0, 0)),
            pl.BlockSpec((1, 2 * H), lambda i: (0, 0)),
        ],
        out_specs=[
            pl.BlockSpec((BR, 2 * H), lambda i: (i, 0)),
            pl.BlockSpec((8, 2 * H), lambda i: (0, 0)),
        ],
        out_shape=[
            jax.ShapeDtypeStruct((E2, 2 * H), jnp.float32),
            jax.ShapeDtypeStruct((8, 2 * H), jnp.float32),
        ],
    )(pre128, b1row, w2bd, b2row)


def _cat_mm(S1, S2, w, brow, deg, vrow):
    BR, NB = 2000, N // 2000
    MH = w.shape[1]

    def body(s1_ref, s2_ref, w_ref, b_ref, d_ref, v_ref, r_ref, st_ref):
        x1 = s1_ref[0] + s1_ref[1]
        x2 = s2_ref[0] + s2_ref[1]
        xcat = jnp.concatenate([x1, x2], axis=1)
        r = jnp.maximum(
            jnp.dot(xcat, w_ref[...], preferred_element_type=jnp.float32)
            + d_ref[...] * v_ref[...] + b_ref[...], 0.0)
        r_ref[...] = r

        @pl.when(pl.program_id(0) == 0)
        def _():
            st_ref[...] = jnp.zeros_like(st_ref)

        st_ref[0:1, :] += jnp.sum(r, axis=0, keepdims=True)
        st_ref[1:2, :] += jnp.sum(r * r, axis=0, keepdims=True)

    return pl.pallas_call(
        body,
        grid=(NB,),
        in_specs=[
            pl.BlockSpec((2, BR, H), lambda j: (0, j, 0)),
            pl.BlockSpec((2, BR, H), lambda j: (0, j, 0)),
            pl.BlockSpec((2 * H, MH), lambda j: (0, 0)),
            pl.BlockSpec((1, MH), lambda j: (0, 0)),
            pl.BlockSpec((BR, 1), lambda j: (j, 0)),
            pl.BlockSpec((1, MH), lambda j: (0, 0)),
        ],
        out_specs=[
            pl.BlockSpec((BR, MH), lambda j: (j, 0)),
            pl.BlockSpec((8, MH), lambda j: (0, 0)),
        ],
        out_shape=[
            jax.ShapeDtypeStruct((N, MH), jnp.float32),
            jax.ShapeDtypeStruct((8, MH), jnp.float32),
        ],
    )(S1, S2, w, brow, deg, vrow)


def _node_mm(xin, w, brow):
    BR, NB = 2000, N // 2000
    K, M = w.shape

    def body(x_ref, w_ref, b_ref, r_ref, st_ref):
        r = jnp.maximum(
            jnp.dot(x_ref[...], w_ref[...],
                    preferred_element_type=jnp.float32) + b_ref[...], 0.0)
        r_ref[...] = r

        @pl.when(pl.program_id(0) == 0)
        def _():
            st_ref[...] = jnp.zeros_like(st_ref)

        st_ref[0:1, :] += jnp.sum(r, axis=0, keepdims=True)
        st_ref[1:2, :] += jnp.sum(r * r, axis=0, keepdims=True)

    return pl.pallas_call(
        body,
        grid=(NB,),
        in_specs=[
            pl.BlockSpec((BR, K), lambda j: (j, 0)),
            pl.BlockSpec((K, M), lambda j: (0, 0)),
            pl.BlockSpec((1, M), lambda j: (0, 0)),
        ],
        out_specs=[
            pl.BlockSpec((BR, M), lambda j: (j, 0)),
            pl.BlockSpec((8, M), lambda j: (0, 0)),
        ],
        out_shape=[
            jax.ShapeDtypeStruct((N, M), jnp.float32),
            jax.ShapeDtypeStruct((8, M), jnp.float32),
        ],
    )(xin, w, brow)


def _final_mm(xin, w, brow):
    BR, NB = 2000, N // 2000
    K, M = w.shape

    def body(x_ref, w_ref, b_ref, o_ref):
        z = jnp.dot(x_ref[...], w_ref[...],
                    preferred_element_type=jnp.float32) + b_ref[...]
        m = jnp.max(z, axis=1, keepdims=True)
        lse = jnp.log(jnp.sum(jnp.exp(z - m), axis=1, keepdims=True)) + m
        o_ref[...] = z - lse

    return pl.pallas_call(
        body,
        grid=(NB,),
        in_specs=[
            pl.BlockSpec((BR, K), lambda j: (j, 0)),
            pl.BlockSpec((K, M), lambda j: (0, 0)),
            pl.BlockSpec((1, M), lambda j: (0, 0)),
        ],
        out_specs=pl.BlockSpec((BR, M), lambda j: (j, 0)),
        out_shape=jax.ShapeDtypeStruct((N, M), jnp.float32),
    )(xin, w, brow)


def _bn_fold(st, g, be):
    mu = st[0] / E
    var = st[1] / E - mu * mu
    s = g / jnp.sqrt(var + EPS)
    return mu, s, be - s * mu


def _bn_fold_n(st, g, be):
    mu = st[0] / N
    var = st[1] / N - mu * mu
    s = g / jnp.sqrt(var + EPS)
    return mu, s, be - s * mu


def _edge_layer(xin_T, src, dst, b1, g1, be1, W2, b2, g2, be2, with_deg):
    pre = _sc_gather(xin_T, src, dst)
    pre128 = pre.reshape(E // 2, 2 * H)
    b1r = jnp.tile(b1, 2).reshape(1, 2 * H)
    st1p = _edge_stats(pre128, b1r)
    st1 = st1p[:, :H] + st1p[:, H:]
    _, s1, c1 = _bn_fold(st1, g1, be1)
    w2p = s1[:, None] * W2
    b2p = c1 @ W2 + b2
    w2bd = jnp.zeros((2 * H, 2 * H), jnp.float32)
    w2bd = w2bd.at[:H, :H].set(w2p).at[H:, H:].set(w2p)
    h2_128, st2p = _edge_mm(pre128, b1r, w2bd, jnp.tile(b2p, 2).reshape(1, -1))
    st2 = st2p[:, :H] + st2p[:, H:]
    mu2 = st2[0] / E
    var2 = st2[1] / E - mu2 * mu2
    s2 = g2 / jnp.sqrt(var2 + EPS)
    c2 = be2 - s2 * mu2
    out = _sc_scatter(h2_128.reshape(E, H), dst, with_deg)
    if with_deg:
        S, degp = out
        return S, s2, c2, degp
    return out, s2, c2


def kernel(x, edge_index, c1_W1, c1_b1, c1_g1, c1_be1, c1_W2, c1_b2, c1_g2,
           c1_be2, c2_W1, c2_b1, c2_g1, c2_be1, c2_W2, c2_b2, c2_g2, c2_be2,
           l1_W, l1_b, l1_g, l1_be, m1_W, m1_b, m1_g, m1_be, m2_W, m2_b,
           m2_g, m2_be, f_W, f_b):
    src = edge_index[0]
    dst = edge_index[1]

    wcat1 = jnp.stack([c1_W1[:D] - c1_W1[D:], c1_W1[D:]])
    T1 = _prep1(x, wcat1)
    S1, s2a, c2a, degp = _edge_layer(T1, src, dst, c1_b1, c1_g1, c1_be1,
                                     c1_W2, c1_b2, c1_g2, c1_be2, True)
    deg = (degp[0, :, 0] + degp[1, :, 0]).reshape(N, 1)

    w2stack = jnp.stack([c2_W1[:H] - c2_W1[H:], c2_W1[H:]])
    wcat2 = s2a[None, :, None] * w2stack
    vp2 = jnp.einsum('h,phk->pk', c2a, w2stack).reshape(2, 1, H)
    T2 = _prep2(S1, wcat2, deg, vp2)
    S2, s2b, c2b = _edge_layer(T2, src, dst, c2_b1, c2_g1, c2_be1,
                               c2_W2, c2_b2, c2_g2, c2_be2, False)

    l1_eff = jnp.concatenate(
        [s2a[:, None] * l1_W[:H], s2b[:, None] * l1_W[H:]], axis=0)
    vcat = c2a @ l1_W[:H] + c2b @ l1_W[H:]
    r1, stA = _cat_mm(S1, S2, l1_eff, l1_b.reshape(1, -1), deg,
                      vcat.reshape(1, -1))
    _, sA, cA = _bn_fold_n(stA, l1_g, l1_be)

    r2, stB = _node_mm(r1, sA[:, None] * m1_W,
                       (cA @ m1_W + m1_b).reshape(1, -1))
    _, sB, cB = _bn_fold_n(stB, m1_g, m1_be)

    r3, stC = _node_mm(r2, sB[:, None] * m2_W,
                       (cB @ m2_W + m2_b).reshape(1, -1))
    _, sC, cC = _bn_fold_n(stC, m2_g, m2_be)

    fw = sC[:, None] * f_W
    fb = cC @ f_W + f_b
    C = f_W.shape[1]
    CP = 16
    fw_pad = jnp.pad(fw, ((0, 0), (0, CP - C)))
    fb_pad = jnp.pad(fb, (0, CP - C), constant_values=-1e30)
    out = _final_mm(r3, fw_pad, fb_pad.reshape(1, CP))
    return out[:, :C]


# speedup vs baseline: 6.9719x; 1.3029x over previous
import functools

import jax
import jax.numpy as jnp
from jax import lax
from jax.experimental import pallas as pl
from jax.experimental.pallas import tpu as pltpu
from jax.experimental.pallas import tpu_sc as plsc

N = 10000
E = 320000
D = 128
H = 64
EPS = 1e-5

NC = 2
NS = 16
NW = NC * NS
CH = 128
EPW = E // NW
NFULL = EPW // CH
TAIL = EPW - NFULL * CH

_MESH = dict(core_axis_name="c", subcore_axis_name="s", num_cores=NC,
             num_subcores=NS)

ROWS_PER_SUB = N // NS


def _sc_gather(T, src, dst):
    @functools.partial(
        pl.kernel,
        out_type=jax.ShapeDtypeStruct((E, H), jnp.float32),
        mesh=plsc.VectorSubcoreMesh(**_MESH),
        scratch_types=[
            pltpu.VMEM((EPW,), jnp.int32),
            pltpu.VMEM((EPW,), jnp.int32),
            pltpu.VMEM((CH, H), jnp.float32),
            pltpu.VMEM((CH, H), jnp.float32),
            pltpu.VMEM((TAIL, H), jnp.float32),
            pltpu.SemaphoreType.DMA,
            pltpu.SemaphoreType.DMA,
        ],
        compiler_params=pltpu.CompilerParams(use_tc_tiling_on_sc=False),
    )
    def k(t_hbm, src_hbm, dst_hbm, pre_hbm, idxd, idxs, buf0, buf1, buf_t,
          sem0, sem1):
        c = lax.axis_index("c")
        s = lax.axis_index("s")
        wid = s * NC + c
        base0 = wid * EPW

        pltpu.sync_copy(dst_hbm.at[pl.ds(base0, EPW)], idxd)
        pltpu.sync_copy(src_hbm.at[pl.ds(base0, EPW)], idxs)

        def shift(i, carry):
            sl = pl.ds(i * 16, 16)
            idxs[sl] = idxs[sl] + N
            return carry

        lax.fori_loop(0, EPW // 16, shift, 0)

        def gsta(ci, b, sem, add):
            iref = idxs if add else idxd
            pltpu.async_copy(t_hbm.at[iref.at[pl.ds(ci * CH, CH)]], b, sem,
                             add=add)

        def gwait(ci, b, sem, add):
            iref = idxs if add else idxd
            pltpu.make_async_copy(t_hbm.at[iref.at[pl.ds(ci * CH, CH)]], b,
                                  sem).wait()

        def ssta(ci, b, sem):
            pltpu.async_copy(b, pre_hbm.at[pl.ds(base0 + ci * CH, CH)], sem)

        def swait(ci, b, sem):
            pltpu.make_async_copy(b, pre_hbm.at[pl.ds(base0 + ci * CH, CH)],
                                  sem).wait()

        NP = NFULL // 2
        gsta(0, buf0, sem0, False)

        def body(i, carry):
            c0 = 2 * i
            c1 = c0 + 1
            gwait(c0, buf0, sem0, False)
            gsta(c0, buf0, sem0, True)

            @pl.when(i > 0)
            def _():
                swait(c1 - 2, buf1, sem1)

            gsta(c1, buf1, sem1, False)
            gwait(c0, buf0, sem0, True)
            ssta(c0, buf0, sem0)
            gwait(c1, buf1, sem1, False)
            gsta(c1, buf1, sem1, True)
            swait(c0, buf0, sem0)

            @pl.when(i < NP - 1)
            def _():
                gsta(c0 + 2, buf0, sem0, False)

            gwait(c1, buf1, sem1, True)
            ssta(c1, buf1, sem1)
            return carry

        lax.fori_loop(0, NP, body, 0)
        swait(NFULL - 1, buf1, sem1)

        tb = NFULL * CH
        pltpu.sync_copy(t_hbm.at[idxd.at[pl.ds(tb, TAIL)]], buf_t)
        pltpu.sync_copy(t_hbm.at[idxs.at[pl.ds(tb, TAIL)]], buf_t, add=True)
        pltpu.sync_copy(buf_t, pre_hbm.at[pl.ds(base0 + tb, TAIL)])

    return k(T, src, dst)


DW = 16


def _sc_scatter(h2, dst, with_deg):
    out_type = [jax.ShapeDtypeStruct((NC, N, H), jnp.float32)]
    scratch = [
        pltpu.VMEM_SHARED((N, H), jnp.float32),
        pltpu.VMEM((ROWS_PER_SUB, H), jnp.float32),
        pltpu.VMEM((CH,), jnp.int32),
        pltpu.VMEM((CH,), jnp.int32),
        pltpu.VMEM((CH, H), jnp.float32),
        pltpu.VMEM((CH, H), jnp.float32),
        pltpu.VMEM((TAIL,), jnp.int32),
        pltpu.VMEM((TAIL, H), jnp.float32),
        pltpu.SemaphoreType.DMA,
        pltpu.SemaphoreType.DMA,
        pltpu.SemaphoreType.DMA,
        pltpu.SemaphoreType.DMA,
    ]
    if with_deg:
        out_type.append(jax.ShapeDtypeStruct((NC, N, DW), jnp.float32))
        scratch += [
            pltpu.VMEM_SHARED((N, DW), jnp.float32),
            pltpu.VMEM((ROWS_PER_SUB, DW), jnp.float32),
            pltpu.VMEM((CH, DW), jnp.float32),
            pltpu.VMEM((TAIL, DW), jnp.float32),
        ]

    @functools.partial(
        pl.kernel,
        out_type=tuple(out_type),
        mesh=plsc.VectorSubcoreMesh(**_MESH),
        scratch_types=scratch,
        compiler_params=pltpu.CompilerParams(use_tc_tiling_on_sc=False),
    )
    def k(h2_hbm, dst_hbm, out_hbm, *rest):
        if with_deg:
            (outd_hbm, acc, stage, idx0, idx1, buf0, buf1, idx_t, buf_t,
             semL0, semL1, semW0, semW1, dacc, dstage, ones, ones_t) = rest
        else:
            (acc, stage, idx0, idx1, buf0, buf1, idx_t, buf_t,
             semL0, semL1, semW0, semW1) = rest
        c = lax.axis_index("c")
        s = lax.axis_index("s")
        wid = s * NC + c
        base0 = wid * EPW
        rsl = pl.ds(s * ROWS_PER_SUB, ROWS_PER_SUB)

        def zrow(i, carry):
            for j in range(H // 16):
                stage[i, pl.ds(j * 16, 16)] = jnp.zeros((16,), jnp.float32)
            return carry

        lax.fori_loop(0, ROWS_PER_SUB, zrow, 0)
        pltpu.sync_copy(stage, acc.at[rsl])
        if with_deg:
            def zdrow(i, carry):
                dstage[i, pl.ds(0, 16)] = jnp.zeros((16,), jnp.float32)
                return carry

            lax.fori_loop(0, ROWS_PER_SUB, zdrow, 0)
            pltpu.sync_copy(dstage, dacc.at[rsl])

            def orow(i, carry):
                ones[i, pl.ds(0, 16)] = jnp.ones((16,), jnp.float32)
                return carry

            lax.fori_loop(0, CH, orow, 0)
            for i in range(TAIL):
                ones_t[i, pl.ds(0, 16)] = jnp.ones((16,), jnp.float32)
        plsc.subcore_barrier()

        def lsta(ci, i_r, b, sem):
            base = base0 + ci * CH
            pltpu.async_copy(dst_hbm.at[pl.ds(base, CH)], i_r, sem)
            pltpu.async_copy(h2_hbm.at[pl.ds(base, CH)], b, sem)

        def lwait(ci, i_r, b, sem):
            base = base0 + ci * CH
            pltpu.make_async_copy(dst_hbm.at[pl.ds(base, CH)], i_r,
                                  sem).wait()
            pltpu.make_async_copy(h2_hbm.at[pl.ds(base, CH)], b, sem).wait()

        def wsta(i_r, b, sem):
            pltpu.async_copy(b, acc.at[i_r], sem, add=True)

        def wwait(i_r, b, sem):
            pltpu.make_async_copy(b, acc.at[i_r], sem).wait()

        NP = NFULL // 2
        lsta(0, idx0, buf0, semL0)

        def body(i, carry):
            c0 = 2 * i
            c1 = c0 + 1
            lwait(c0, idx0, buf0, semL0)

            @pl.when(i > 0)
            def _():
                wwait(idx1, buf1, semW1)

            lsta(c1, idx1, buf1, semL1)
            wsta(idx0, buf0, semW0)
            if with_deg:
                pltpu.sync_copy(ones, dacc.at[idx0], add=True)
            lwait(c1, idx1, buf1, semL1)
            wwait(idx0, buf0, semW0)

            @pl.when(i < NP - 1)
            def _():
                lsta(c0 + 2, idx0, buf0, semL0)

            wsta(idx1, buf1, semW1)
            if with_deg:
                pltpu.sync_copy(ones, dacc.at[idx1], add=True)
            return carry

        lax.fori_loop(0, NP, body, 0)
        wwait(idx1, buf1, semW1)

        tb = base0 + NFULL * CH
        pltpu.sync_copy(dst_hbm.at[pl.ds(tb, TAIL)], idx_t)
        pltpu.sync_copy(h2_hbm.at[pl.ds(tb, TAIL)], buf_t)
        pltpu.sync_copy(buf_t, acc.at[idx_t], add=True)
        if with_deg:
            pltpu.sync_copy(ones_t, dacc.at[idx_t], add=True)

        plsc.subcore_barrier()
        pltpu.sync_copy(acc.at[rsl], stage)
        pltpu.sync_copy(stage, out_hbm.at[c, rsl])
        if with_deg:
            pltpu.sync_copy(dacc.at[rsl], dstage)
            pltpu.sync_copy(dstage, outd_hbm.at[c, rsl])

    res = k(h2, dst)
    return res if with_deg else res[0]


def _prep1(x, wcat):
    BR, NB = 2000, N // 2000

    def body(x_ref, w_ref, o_ref):
        o_ref[...] = jnp.dot(x_ref[...], w_ref[0],
                             preferred_element_type=jnp.float32)

    return pl.pallas_call(
        body,
        grid=(2, NB),
        in_specs=[
            pl.BlockSpec((BR, D), lambda p, j: (j, 0)),
            pl.BlockSpec((1, D, H), lambda p, j: (p, 0, 0)),
        ],
        out_specs=pl.BlockSpec((BR, H), lambda p, j: (p * NB + j, 0)),
        out_shape=jax.ShapeDtypeStruct((2 * N, H), jnp.float32),
    )(x, wcat)


def _prep2(S, wcat, deg, vp):
    BR, NB = 2000, N // 2000

    def body(s_ref, w_ref, d_ref, v_ref, o_ref):
        xin = s_ref[0] + s_ref[1]
        o_ref[...] = (jnp.dot(xin, w_ref[0],
                              preferred_element_type=jnp.float32)
                      + d_ref[...] * v_ref[0])

    return pl.pallas_call(
        body,
        grid=(2, NB),
        in_specs=[
            pl.BlockSpec((2, BR, H), lambda p, j: (0, j, 0)),
            pl.BlockSpec((1, H, H), lambda p, j: (p, 0, 0)),
            pl.BlockSpec((BR, 1), lambda p, j: (j, 0)),
            pl.BlockSpec((1, 1, H), lambda p, j: (p, 0, 0)),
        ],
        out_specs=pl.BlockSpec((BR, H), lambda p, j: (p * NB + j, 0)),
        out_shape=jax.ShapeDtypeStruct((2 * N, H), jnp.float32),
    )(S, wcat, deg, vp)


def _edge_stats(pre128, b1row):
    BR = 8000
    E2 = E // 2
    NB = E2 // BR

    def body(p_ref, b_ref, o_ref):
        h = jnp.maximum(p_ref[...] + b_ref[...], 0.0)

        @pl.when(pl.program_id(0) == 0)
        def _():
            o_ref[...] = jnp.zeros_like(o_ref)

        o_ref[0:1, :] += jnp.sum(h, axis=0, keepdims=True)
        o_ref[1:2, :] += jnp.sum(h * h, axis=0, keepdims=True)

    return pl.pallas_call(
        body,
        grid=(NB,),
        in_specs=[
            pl.BlockSpec((BR, 2 * H), lambda i: (i, 0)),
            pl.BlockSpec((1, 2 * H), lambda i: (0, 0)),
        ],
        out_specs=pl.BlockSpec((8, 2 * H), lambda i: (0, 0)),
        out_shape=jax.ShapeDtypeStruct((8, 2 * H), jnp.float32),
    )(pre128, b1row)


def _edge_mm(pre128, b1row, w2bd, b2row):
    BR = 8000
    E2 = E // 2
    NB = E2 // BR

    def body(p_ref, b1_ref, w_ref, b2_ref, h2_ref, st_ref):
        h1 = jnp.maximum(p_ref[...] + b1_ref[...], 0.0)
        h2 = jnp.maximum(
            jnp.dot(h1, w_ref[...], preferred_element_type=jnp.float32)
            + b2_ref[...], 0.0)
        h2_ref[...] = h2

        @pl.when(pl.program_id(0) == 0)
        def _():
            st_ref[...] = jnp.zeros_like(st_ref)

        st_ref[0:1, :] += jnp.sum(h2, axis=0, keepdims=True)
        st_ref[1:2, :] += jnp.sum(h2 * h2, axis=0, keepdims=True)

    return pl.pallas_call(
        body,
        grid=(NB,),
        in_specs=[
            pl.BlockSpec((BR, 2 * H), lambda i: (i, 0)),
            pl.BlockSpec((1, 2 * H), lambda i: (0, 0)),
            pl.BlockSpec((2 * H, 2 * H), lambda i: (0, 0)),
            pl.BlockSpec((1, 2 * H), lambda i: (0, 0)),
        ],
        out_specs=[
            pl.BlockSpec((BR, 2 * H), lambda i: (i, 0)),
            pl.BlockSpec((8, 2 * H), lambda i: (0, 0)),
        ],
        out_shape=[
            jax.ShapeDtypeStruct((E2, 2 * H), jnp.float32),
            jax.ShapeDtypeStruct((8, 2 * H), jnp.float32),
        ],
    )(pre128, b1row, w2bd, b2row)


def _cat_mm(S1, S2, w, brow, deg, vrow):
    BR, NB = 2000, N // 2000
    MH = w.shape[1]

    def body(s1_ref, s2_ref, w_ref, b_ref, d_ref, v_ref, r_ref, st_ref):
        x1 = s1_ref[0] + s1_ref[1]
        x2 = s2_ref[0] + s2_ref[1]
        xcat = jnp.concatenate([x1, x2], axis=1)
        r = jnp.maximum(
            jnp.dot(xcat, w_ref[...], preferred_element_type=jnp.float32)
            + d_ref[...] * v_ref[...] + b_ref[...], 0.0)
        r_ref[...] = r

        @pl.when(pl.program_id(0) == 0)
        def _():
            st_ref[...] = jnp.zeros_like(st_ref)

        st_ref[0:1, :] += jnp.sum(r, axis=0, keepdims=True)
        st_ref[1:2, :] += jnp.sum(r * r, axis=0, keepdims=True)

    return pl.pallas_call(
        body,
        grid=(NB,),
        in_specs=[
            pl.BlockSpec((2, BR, H), lambda j: (0, j, 0)),
            pl.BlockSpec((2, BR, H), lambda j: (0, j, 0)),
            pl.BlockSpec((2 * H, MH), lambda j: (0, 0)),
            pl.BlockSpec((1, MH), lambda j: (0, 0)),
            pl.BlockSpec((BR, 1), lambda j: (j, 0)),
            pl.BlockSpec((1, MH), lambda j: (0, 0)),
        ],
        out_specs=[
            pl.BlockSpec((BR, MH), lambda j: (j, 0)),
            pl.BlockSpec((8, MH), lambda j: (0, 0)),
        ],
        out_shape=[
            jax.ShapeDtypeStruct((N, MH), jnp.float32),
            jax.ShapeDtypeStruct((8, MH), jnp.float32),
        ],
    )(S1, S2, w, brow, deg, vrow)


def _node_mm(xin, w, brow):
    BR, NB = 2000, N // 2000
    K, M = w.shape

    def body(x_ref, w_ref, b_ref, r_ref, st_ref):
        r = jnp.maximum(
            jnp.dot(x_ref[...], w_ref[...],
                    preferred_element_type=jnp.float32) + b_ref[...], 0.0)
        r_ref[...] = r

        @pl.when(pl.program_id(0) == 0)
        def _():
            st_ref[...] = jnp.zeros_like(st_ref)

        st_ref[0:1, :] += jnp.sum(r, axis=0, keepdims=True)
        st_ref[1:2, :] += jnp.sum(r * r, axis=0, keepdims=True)

    return pl.pallas_call(
        body,
        grid=(NB,),
        in_specs=[
            pl.BlockSpec((BR, K), lambda j: (j, 0)),
            pl.BlockSpec((K, M), lambda j: (0, 0)),
            pl.BlockSpec((1, M), lambda j: (0, 0)),
        ],
        out_specs=[
            pl.BlockSpec((BR, M), lambda j: (j, 0)),
            pl.BlockSpec((8, M), lambda j: (0, 0)),
        ],
        out_shape=[
            jax.ShapeDtypeStruct((N, M), jnp.float32),
            jax.ShapeDtypeStruct((8, M), jnp.float32),
        ],
    )(xin, w, brow)


def _final_mm(xin, w, brow):
    BR, NB = 2000, N // 2000
    K, M = w.shape

    def body(x_ref, w_ref, b_ref, o_ref):
        z = jnp.dot(x_ref[...], w_ref[...],
                    preferred_element_type=jnp.float32) + b_ref[...]
        m = jnp.max(z, axis=1, keepdims=True)
        lse = jnp.log(jnp.sum(jnp.exp(z - m), axis=1, keepdims=True)) + m
        o_ref[...] = z - lse

    return pl.pallas_call(
        body,
        grid=(NB,),
        in_specs=[
            pl.BlockSpec((BR, K), lambda j: (j, 0)),
            pl.BlockSpec((K, M), lambda j: (0, 0)),
            pl.BlockSpec((1, M), lambda j: (0, 0)),
        ],
        out_specs=pl.BlockSpec((BR, M), lambda j: (j, 0)),
        out_shape=jax.ShapeDtypeStruct((N, M), jnp.float32),
    )(xin, w, brow)


def _bn_fold(st, g, be):
    mu = st[0] / E
    var = st[1] / E - mu * mu
    s = g / jnp.sqrt(var + EPS)
    return mu, s, be - s * mu


def _bn_fold_n(st, g, be):
    mu = st[0] / N
    var = st[1] / N - mu * mu
    s = g / jnp.sqrt(var + EPS)
    return mu, s, be - s * mu


def _edge_layer(xin_T, src, dst, b1, g1, be1, W2, b2, g2, be2, with_deg):
    pre = _sc_gather(xin_T, src, dst)
    pre128 = pre.reshape(E // 2, 2 * H)
    b1r = jnp.tile(b1, 2).reshape(1, 2 * H)
    st1p = _edge_stats(pre128, b1r)
    st1 = st1p[:, :H] + st1p[:, H:]
    _, s1, c1 = _bn_fold(st1, g1, be1)
    w2p = s1[:, None] * W2
    b2p = c1 @ W2 + b2
    w2bd = jnp.zeros((2 * H, 2 * H), jnp.float32)
    w2bd = w2bd.at[:H, :H].set(w2p).at[H:, H:].set(w2p)
    h2_128, st2p = _edge_mm(pre128, b1r, w2bd, jnp.tile(b2p, 2).reshape(1, -1))
    st2 = st2p[:, :H] + st2p[:, H:]
    mu2 = st2[0] / E
    var2 = st2[1] / E - mu2 * mu2
    s2 = g2 / jnp.sqrt(var2 + EPS)
    c2 = be2 - s2 * mu2
    out = _sc_scatter(h2_128.reshape(E, H), dst, with_deg)
    if with_deg:
        S, degp = out
        return S, s2, c2, degp
    return out, s2, c2


def kernel(x, edge_index, c1_W1, c1_b1, c1_g1, c1_be1, c1_W2, c1_b2, c1_g2,
           c1_be2, c2_W1, c2_b1, c2_g1, c2_be1, c2_W2, c2_b2, c2_g2, c2_be2,
           l1_W, l1_b, l1_g, l1_be, m1_W, m1_b, m1_g, m1_be, m2_W, m2_b,
           m2_g, m2_be, f_W, f_b):
    src = edge_index[0]
    dst = edge_index[1]

    wcat1 = jnp.stack([c1_W1[:D] - c1_W1[D:], c1_W1[D:]])
    T1 = _prep1(x, wcat1)
    S1, s2a, c2a, degp = _edge_layer(T1, src, dst, c1_b1, c1_g1, c1_be1,
                                     c1_W2, c1_b2, c1_g2, c1_be2, True)
    deg = (degp[0, :, 0] + degp[1, :, 0]).reshape(N, 1)

    w2stack = jnp.stack([c2_W1[:H] - c2_W1[H:], c2_W1[H:]])
    wcat2 = s2a[None, :, None] * w2stack
    vp2 = jnp.einsum('h,phk->pk', c2a, w2stack).reshape(2, 1, H)
    T2 = _prep2(S1, wcat2, deg, vp2)
    S2, s2b, c2b = _edge_layer(T2, src, dst, c2_b1, c2_g1, c2_be1,
                               c2_W2, c2_b2, c2_g2, c2_be2, False)

    l1_eff = jnp.concatenate(
        [s2a[:, None] * l1_W[:H], s2b[:, None] * l1_W[H:]], axis=0)
    vcat = c2a @ l1_W[:H] + c2b @ l1_W[H:]
    r1, stA = _cat_mm(S1, S2, l1_eff, l1_b.reshape(1, -1), deg,
                      vcat.reshape(1, -1))
    _, sA, cA = _bn_fold_n(stA, l1_g, l1_be)

    r2, stB = _node_mm(r1, sA[:, None] * m1_W,
                       (cA @ m1_W + m1_b).reshape(1, -1))
    _, sB, cB = _bn_fold_n(stB, m1_g, m1_be)

    r3, stC = _node_mm(r2, sB[:, None] * m2_W,
                       (cB @ m2_W + m2_b).reshape(1, -1))
    _, sC, cC = _bn_fold_n(stC, m2_g, m2_be)

    fw = sC[:, None] * f_W
    fb = cC @ f_W + f_b
    C = f_W.shape[1]
    CP = 16
    fw_pad = jnp.pad(fw, ((0, 0), (0, CP - C)))
    fb_pad = jnp.pad(fb, (0, CP - C), constant_values=-1e30)
    out = _final_mm(r3, fw_pad, fb_pad.reshape(1, CP))
    return out[:, :C]
